# baseline (speedup 1.0000x reference)
import jax
import jax.numpy as jnp
from jax import lax
from jax.experimental import pallas as pl
from jax.experimental.pallas import tpu as pltpu
from jax.experimental.pallas import tpu_sc as plsc

_CAT_IDX = (2, 4, 6, 8, 10, 12, 14, 16, 18, 20)
_CONT_IDX = tuple(n for n in range(21) if n not in _CAT_IDX)
_N = 21
_H = 16
_L = 16
_TOK = 16384
_NC = 2
_NS = 16
_TPW = 1024
_ST = 128
_NSTG = _TPW // _ST
_NTGS = _ST // _L
_OSZ = _N * _ST * 8
_EMB0 = 0
_W0 = len(_CAT_IDX) * 2 * _H
_B0 = _W0 + len(_CONT_IDX) * _H
_WLEN = _B0 + len(_CONT_IDX) * _H


def _sc_body(x_hbm, wts_hbm, out_hbm,
             x_v0, x_v1, o_v0, o_v1, wts_v, sx0, sx1, so0, so1):
    cid = lax.axis_index("c")
    sid = lax.axis_index("s")
    wid = sid * _NC + cid
    kk = wid & 1
    tb = wid >> 1
    k8 = kk * 8
    t0 = tb * _TPW

    pltpu.sync_copy(wts_hbm, wts_v)

    lanes = jnp.arange(_L, dtype=jnp.int32)
    piece_w = (kk * 128 + tb * 8) * 1024

    def bcast(off, k):
        row = wts_v[pl.ds(off, _H)]
        return jnp.take_along_axis(
            row, jnp.full((_L,), k8 + k, jnp.int32), axis=0,
            mode="promise_in_bounds")

    def compute_stage(x_v, o_v):
        for n in range(_N):
            is_cat = n in _CAT_IDX
            if is_cat:
                ci = _CAT_IDX.index(n)
                e0b = [bcast(_EMB0 + ci * 2 * _H, k) for k in range(8)]
                e1b = [bcast(_EMB0 + ci * 2 * _H + _H, k) for k in range(8)]
            else:
                li = _CONT_IDX.index(n)
                wb = [bcast(_W0 + li * _H, k) for k in range(8)]
                bb = [bcast(_B0 + li * _H, k) for k in range(8)]
            col = jnp.full((_L,), n, jnp.int32)
            for tgs in range(_NTGS):
                xv = plsc.load_gather(x_v, [tgs * _L + lanes, col])
                off = n * 1024 + tgs * _L
                if is_cat:
                    m = xv >= 1.0
                    for k in range(8):
                        o_v[pl.ds(off + k * 128, _L)] = (
                            jnp.where(m, e1b[k], e0b[k]))
                else:
                    for k in range(8):
                        o_v[pl.ds(off + k * 128, _L)] = xv * wb[k] + bb[k]

    pltpu.async_copy(x_hbm.at[pl.ds(t0, _ST)], x_v0, sx0)
    pltpu.async_copy(x_hbm.at[pl.ds(t0 + _ST, _ST)], x_v1, sx1)

    def pair(i, _):
        for half, (x_v, o_v, sx, so) in enumerate(
                ((x_v0, o_v0, sx0, so0), (x_v1, o_v1, sx1, so1))):
            s = i * 2 + half
            pltpu.make_async_copy(x_hbm.at[pl.ds(0, _ST)], x_v, sx).wait()
            @pl.when(i >= 1)
            def _():
                pltpu.make_async_copy(
                    out_hbm.at[pl.ds(0, _OSZ)], o_v, so).wait()

            compute_stage(x_v, o_v)

            base = piece_w + s * 1024
            for n in range(_N):
                pltpu.async_copy(
                    o_v.at[pl.ds(n * 1024, 1024)],
                    out_hbm.at[pl.ds(n * (_H * _TOK) + base, 1024)], so)
            @pl.when(s + 2 < _NSTG)
            def _():
                pltpu.async_copy(
                    x_hbm.at[pl.ds(t0 + (s + 2) * _ST, _ST)], x_v, sx)
        return 0

    lax.fori_loop(0, _NSTG // 2, pair, 0)
    pltpu.make_async_copy(out_hbm.at[pl.ds(0, _OSZ)], o_v0, so0).wait()
    pltpu.make_async_copy(out_hbm.at[pl.ds(0, _OSZ)], o_v1, so1).wait()


@jax.jit
def _run(x, emb_tables, lin_W, lin_b):
    wts = jnp.concatenate(
        [emb_tables.reshape(-1), lin_W.reshape(-1), lin_b.reshape(-1)])
    mesh = plsc.VectorSubcoreMesh(core_axis_name="c", subcore_axis_name="s")
    f = pl.kernel(
        _sc_body,
        out_type=jax.ShapeDtypeStruct((_TOK * _N * _H,), jnp.float32),
        mesh=mesh,
        compiler_params=pltpu.CompilerParams(needs_layout_passes=False),
        scratch_types=[
            pltpu.VMEM((_ST, _N), jnp.float32),
            pltpu.VMEM((_ST, _N), jnp.float32),
            pltpu.VMEM((_OSZ,), jnp.float32),
            pltpu.VMEM((_OSZ,), jnp.float32),
            pltpu.VMEM((_WLEN,), jnp.float32),
            pltpu.SemaphoreType.DMA,
            pltpu.SemaphoreType.DMA,
            pltpu.SemaphoreType.DMA,
            pltpu.SemaphoreType.DMA,
        ],
    )
    out = f(x, wts)
    buf = out.reshape(_N, 2, _TOK // 128, 8, 128)
    return buf.transpose(2, 4, 0, 1, 3).reshape(_TOK, _N, _H)


def kernel(x, cu_seqlens, emb_tables, lin_W, lin_b):
    del cu_seqlens
    return _run(x.astype(jnp.float32), emb_tables, lin_W, lin_b)


# speedup vs baseline: 1.3076x; 1.3076x over previous
import jax
import jax.numpy as jnp
from jax import lax
from jax.experimental import pallas as pl
from jax.experimental.pallas import tpu as pltpu
from jax.experimental.pallas import tpu_sc as plsc

_CAT_IDX = (2, 4, 6, 8, 10, 12, 14, 16, 18, 20)
_CONT_IDX = tuple(n for n in range(21) if n not in _CAT_IDX)
_N = 21
_H = 16
_L = 16
_TOK = 16384
_NC = 2
_NS = 16
_TPW = 1024
_NTG = _TPW // _L
_EMB0 = 0
_W0 = len(_CAT_IDX) * 2 * _H
_B0 = _W0 + len(_CONT_IDX) * _H
_WLEN = _B0 + len(_CONT_IDX) * _H


def _sc_body(x_hbm, wts_hbm, out_hbm, x_v, o_v0, o_v1, wts_v, sem0, sem1):
    cid = lax.axis_index("c")
    sid = lax.axis_index("s")
    wid = sid * _NC + cid
    kk = wid & 1
    tb = wid >> 1
    k8 = kk * 8

    pltpu.sync_copy(wts_hbm, wts_v)
    pltpu.sync_copy(x_hbm.at[pl.ds(tb * (_TPW * _N), _TPW * _N)], x_v)

    lanes = jnp.arange(_L, dtype=jnp.int32)

    o_bufs = (o_v0, o_v1)
    sems = (sem0, sem1)
    pending = [None, None]

    piece_w = (kk * 128 + tb * 8) * 1024

    def bcast(off, k):
        row = wts_v[pl.ds(off, _H)]
        return jnp.take_along_axis(
            row, jnp.full((_L,), k8 + k, jnp.int32), axis=0,
            mode="promise_in_bounds")

    for n in range(_N):
        b = n & 1
        o_v = o_bufs[b]
        if pending[b] is not None:
            pending[b].wait()

        is_cat = n in _CAT_IDX
        if is_cat:
            ci = _CAT_IDX.index(n)
            e0b = [bcast(_EMB0 + ci * 2 * _H, k) for k in range(8)]
            e1b = [bcast(_EMB0 + ci * 2 * _H + _H, k) for k in range(8)]
        else:
            li = _CONT_IDX.index(n)
            wb = [bcast(_W0 + li * _H, k) for k in range(8)]
            bb = [bcast(_B0 + li * _H, k) for k in range(8)]

        base_idx = lanes * _N + n

        def per_tg(tg, _):
            xv = plsc.load_gather(x_v, [base_idx + tg * (_L * _N)])
            off = (tg >> 3) * 1024 + (tg & 7) * _L
            if is_cat:
                m = xv >= 1.0
                for k in range(8):
                    o_v[pl.ds(off + k * 128, _L)] = jnp.where(m, e1b[k], e0b[k])
            else:
                for k in range(8):
                    o_v[pl.ds(off + k * 128, _L)] = xv * wb[k] + bb[k]
            return 0

        lax.fori_loop(0, _NTG, per_tg, 0, unroll=4)

        dst = out_hbm.at[pl.ds(n * (_H * _TOK) + piece_w, 8 * _TPW)]
        pending[b] = pltpu.async_copy(o_v, dst, sems[b])

    pending[0].wait()
    pending[1].wait()


@jax.jit
def _run(x, emb_tables, lin_W, lin_b):
    wts = jnp.concatenate(
        [emb_tables.reshape(-1), lin_W.reshape(-1), lin_b.reshape(-1)])
    mesh = plsc.VectorSubcoreMesh(core_axis_name="c", subcore_axis_name="s")
    f = pl.kernel(
        _sc_body,
        out_type=jax.ShapeDtypeStruct((_TOK * _N * _H,), jnp.float32),
        mesh=mesh,
        compiler_params=pltpu.CompilerParams(needs_layout_passes=False),
        scratch_types=[
            pltpu.VMEM((_TPW * _N,), jnp.float32),
            pltpu.VMEM((8 * _TPW,), jnp.float32),
            pltpu.VMEM((8 * _TPW,), jnp.float32),
            pltpu.VMEM((_WLEN,), jnp.float32),
            pltpu.SemaphoreType.DMA,
            pltpu.SemaphoreType.DMA,
        ],
    )
    out = f(x.reshape(-1), wts)
    buf = out.reshape(_N, 2, _TOK // 128, 8, 128)
    return buf.transpose(2, 4, 0, 1, 3).reshape(_TOK, _N, _H)


def kernel(x, cu_seqlens, emb_tables, lin_W, lin_b):
    del cu_seqlens
    return _run(x.astype(jnp.float32), emb_tables, lin_W, lin_b)


# speedup vs baseline: 1.5229x; 1.1646x over previous
import jax
import jax.numpy as jnp
from jax import lax
from jax.experimental import pallas as pl
from jax.experimental.pallas import tpu as pltpu
from jax.experimental.pallas import tpu_sc as plsc

_CAT_IDX = (2, 4, 6, 8, 10, 12, 14, 16, 18, 20)
_CONT_IDX = tuple(n for n in range(21) if n not in _CAT_IDX)
_N = 21
_H = 16
_L = 16
_TOK = 16384
_NC = 2
_NS = 16
_TPW = 1024
_NTG = _TPW // _L
_EMB0 = 0
_W0 = len(_CAT_IDX) * 2 * _H
_B0 = _W0 + len(_CONT_IDX) * _H
_WLEN = _B0 + len(_CONT_IDX) * _H


def _sc_body(x_hbm, wts_hbm, out_hbm, x_v, o_v0, o_v1, wts_v, sem0, sem1):
    cid = lax.axis_index("c")
    sid = lax.axis_index("s")
    wid = sid * _NC + cid
    kk = wid & 1
    tb = wid >> 1
    k8 = kk * 8

    pltpu.sync_copy(wts_hbm, wts_v)
    pltpu.sync_copy(x_hbm.at[pl.ds(tb * (_TPW * _N), _TPW * _N)], x_v)

    lanes = jnp.arange(_L, dtype=jnp.int32)

    o_bufs = (o_v0, o_v1)
    sems = (sem0, sem1)
    pending = [None, None]

    piece_w = (kk * 128 + tb * 8) * 1024

    def bcast(off, k):
        row = wts_v[pl.ds(off, _H)]
        return jnp.take_along_axis(
            row, jnp.full((_L,), k8 + k, jnp.int32), axis=0,
            mode="promise_in_bounds")

    for n in range(_N):
        b = n & 1
        o_v = o_bufs[b]
        if pending[b] is not None:
            pending[b].wait()

        is_cat = n in _CAT_IDX
        if is_cat:
            ci = _CAT_IDX.index(n)
            e0b = [bcast(_EMB0 + ci * 2 * _H, k) for k in range(8)]
            e1b = [bcast(_EMB0 + ci * 2 * _H + _H, k) for k in range(8)]
        else:
            li = _CONT_IDX.index(n)
            wb = [bcast(_W0 + li * _H, k) for k in range(8)]
            bb = [bcast(_B0 + li * _H, k) for k in range(8)]

        base_idx = lanes * _N + n

        @plsc.parallel_loop(0, _NTG, unroll=2)
        def per_tg(tg):
            xv = plsc.load_gather(x_v, [base_idx + tg * (_L * _N)])
            off = (tg >> 3) * 1024 + (tg & 7) * _L
            if is_cat:
                m = xv >= 1.0
                for k in range(8):
                    o_v[pl.ds(off + k * 128, _L)] = jnp.where(m, e1b[k], e0b[k])
            else:
                for k in range(8):
                    o_v[pl.ds(off + k * 128, _L)] = xv * wb[k] + bb[k]

        dst = out_hbm.at[pl.ds(n * (_H * _TOK) + piece_w, 8 * _TPW)]
        pending[b] = pltpu.async_copy(o_v, dst, sems[b])

    pending[0].wait()
    pending[1].wait()


@jax.jit
def _run(x, emb_tables, lin_W, lin_b):
    wts = jnp.concatenate(
        [emb_tables.reshape(-1), lin_W.reshape(-1), lin_b.reshape(-1)])
    mesh = plsc.VectorSubcoreMesh(core_axis_name="c", subcore_axis_name="s")
    f = pl.kernel(
        _sc_body,
        out_type=jax.ShapeDtypeStruct((_TOK * _N * _H,), jnp.float32),
        mesh=mesh,
        compiler_params=pltpu.CompilerParams(needs_layout_passes=False),
        scratch_types=[
            pltpu.VMEM((_TPW * _N,), jnp.float32),
            pltpu.VMEM((8 * _TPW,), jnp.float32),
            pltpu.VMEM((8 * _TPW,), jnp.float32),
            pltpu.VMEM((_WLEN,), jnp.float32),
            pltpu.SemaphoreType.DMA,
            pltpu.SemaphoreType.DMA,
        ],
    )
    out = f(x.reshape(-1), wts)
    buf = out.reshape(_N, 2, _TOK // 128, 8, 128)
    return buf.transpose(2, 4, 0, 1, 3).reshape(_TOK, _N, _H)


def kernel(x, cu_seqlens, emb_tables, lin_W, lin_b):
    del cu_seqlens
    return _run(x.astype(jnp.float32), emb_tables, lin_W, lin_b)
